# baseline (speedup 1.0000x reference)
import functools

import jax
import jax.numpy as jnp
from jax import lax
from jax.experimental import pallas as pl
from jax.experimental.pallas import tpu as pltpu
from jax.experimental.pallas import tpu_sc as plsc

_N = 10000
_E = 320000
_F = 128
_C = 64

_NC = 2
_NS = 16
_NW = _NC * _NS
_CHUNK = 128
_CPW = 80
_IBLK = 8
_EPAD = _NW * _CPW * _CHUNK
_ACC = 10112
_RPT = _ACC // _NS

_mesh = plsc.VectorSubcoreMesh(core_axis_name="c", subcore_axis_name="s")


_NCHUNK = _EPAD // _CHUNK
_CPT = _NCHUNK // _NS


def _sc_degree_body(idx_hbm, ones_hbm, zeros_hbm, deg_hbm,
                    idx_v, ones_v, acc):
    cid = lax.axis_index("c")
    sid = lax.axis_index("s")
    rows = pl.ds(sid * _RPT, _RPT)
    pltpu.sync_copy(zeros_hbm, acc.at[rows])
    pltpu.sync_copy(ones_hbm, ones_v)
    pltpu.sync_copy(idx_hbm.at[cid, sid], idx_v)
    plsc.subcore_barrier()

    def body(j, carry):
        pltpu.sync_copy(ones_v, acc.at[idx_v.at[j]], add=True)
        return carry

    lax.fori_loop(0, _CPT, body, 0)
    plsc.subcore_barrier()
    pltpu.sync_copy(acc.at[rows], deg_hbm.at[cid, rows])


_sc_degree = pl.kernel(
    _sc_degree_body,
    out_type=jax.ShapeDtypeStruct((_NC, _ACC, _F), jnp.float32),
    mesh=_mesh,
    scratch_types=[
        pltpu.VMEM((_CPT, _CHUNK), jnp.int32),
        pltpu.VMEM((_CHUNK, _F), jnp.float32),
        pltpu.VMEM_SHARED((_ACC, _F), jnp.float32),
    ],
)


def _sc_msg_body(feat, h_hbm, src_hbm, dst_hbm, zeros_hbm, out_hbm,
                 src_v, dst_v, msg_a, sem, acc):
    cid = lax.axis_index("c")
    sid = lax.axis_index("s")
    w = cid * _NS + sid
    rows = pl.ds(sid * _RPT, _RPT)
    pltpu.sync_copy(zeros_hbm, acc.at[rows])
    pltpu.sync_copy(src_hbm.at[w], src_v)
    pltpu.sync_copy(dst_hbm.at[w], dst_v)
    plsc.subcore_barrier()

    def body(j, carry):
        pltpu.async_copy(h_hbm.at[src_v.at[j]], msg_a, sem).wait()
        pltpu.sync_copy(msg_a, acc.at[dst_v.at[j]], add=True)
        return carry

    lax.fori_loop(0, _CPW, body, 0)
    plsc.subcore_barrier()
    pltpu.sync_copy(acc.at[rows], out_hbm.at[cid, rows])


def _make_sc_msg(feat):
    return pl.kernel(
        functools.partial(_sc_msg_body, feat),
        out_type=jax.ShapeDtypeStruct((_NC, _ACC, feat), jnp.float32),
        mesh=_mesh,
        scratch_types=[
            pltpu.VMEM((_CPW, _CHUNK), jnp.int32),
            pltpu.VMEM((_CPW, _CHUNK), jnp.int32),
            pltpu.VMEM((_CHUNK, feat), jnp.float32),
            pltpu.SemaphoreType.DMA,
            pltpu.VMEM_SHARED((_ACC, feat), jnp.float32),
        ],
    )


_sc_msg_h = _make_sc_msg(_F)


_BLK = 1000


def _tc1_body(dout, din, x, w1, h1, no16, ni16):
    no = lax.rsqrt(jnp.maximum(dout[..., :16], 1.0))
    ni = lax.rsqrt(jnp.maximum(din[..., :16], 1.0))
    no16[...] = no
    ni16[...] = ni
    h = jnp.dot(x[...], w1[...], preferred_element_type=jnp.float32,
                precision=lax.Precision.HIGHEST)
    h1[...] = h * no[:, 0:1]


def _tc_layer1(dout, din, x, w1):
    spec16 = pl.BlockSpec((_BLK, 16), lambda i: (i, 0))
    specF = pl.BlockSpec((_BLK, _F), lambda i: (i, 0))
    return pl.pallas_call(
        _tc1_body,
        grid=(_N // _BLK,),
        in_specs=[
            specF, specF, specF,
            pl.BlockSpec((_F, _F), lambda i: (0, 0)),
        ],
        out_specs=[specF, spec16, spec16],
        out_shape=[
            jax.ShapeDtypeStruct((_N, _F), jnp.float32),
            jax.ShapeDtypeStruct((_N, 16), jnp.float32),
            jax.ShapeDtypeStruct((_N, 16), jnp.float32),
        ],
    )(dout, din, x, w1)


def _tc2_body(a0, a1, ni16, b1, w2, no16, h2):
    agg = (a0[...] + a1[...]) * ni16[..., 0:1] + b1[...]
    h = jnp.maximum(agg, 0.0)
    h = jnp.dot(h, w2[...], preferred_element_type=jnp.float32,
                precision=lax.Precision.HIGHEST)
    h = h * no16[..., 0:1]
    h2[...] = jnp.concatenate([h, jnp.zeros_like(h)], axis=1)


def _tc_layer2(a0, a1, ni16, b1, w2, no16):
    spec16 = pl.BlockSpec((_BLK, 16), lambda i: (i, 0))
    return pl.pallas_call(
        _tc2_body,
        grid=(_N // _BLK,),
        in_specs=[
            pl.BlockSpec((_BLK, _F), lambda i: (i, 0)),
            pl.BlockSpec((_BLK, _F), lambda i: (i, 0)),
            spec16,
            pl.BlockSpec((1, _F), lambda i: (0, 0)),
            pl.BlockSpec((_F, _C), lambda i: (0, 0)),
            spec16,
        ],
        out_specs=pl.BlockSpec((_BLK, 2 * _C), lambda i: (i, 0)),
        out_shape=jax.ShapeDtypeStruct((_N, 2 * _C), jnp.float32),
    )(a0, a1, ni16, b1, w2, no16)


def _tc3_body(p0, p1, ni16, b2, out):
    agg = p0[..., :_C] + p1[..., :_C]
    out[...] = agg * ni16[..., 0:1] + b2[...]


def _tc_final(p0, p1, ni16, b2):
    spec16 = pl.BlockSpec((_BLK, 16), lambda i: (i, 0))
    return pl.pallas_call(
        _tc3_body,
        grid=(_N // _BLK,),
        in_specs=[
            pl.BlockSpec((_BLK, _F), lambda i: (i, 0)),
            pl.BlockSpec((_BLK, _F), lambda i: (i, 0)),
            spec16,
            pl.BlockSpec((1, _C), lambda i: (0, 0)),
        ],
        out_specs=pl.BlockSpec((_BLK, _C), lambda i: (i, 0)),
        out_shape=jax.ShapeDtypeStruct((_N, _C), jnp.float32),
    )(p0, p1, ni16, b2)


def kernel(in_feat, edge_index, W1, b1, W2, b2):
    src = edge_index[0].astype(jnp.int32)
    dst = edge_index[1].astype(jnp.int32)
    npad = _EPAD - _E
    shape3 = (_NW, _CPW, _CHUNK)
    src_g = jnp.concatenate([src, jnp.zeros((npad,), jnp.int32)]).reshape(shape3)
    src_d = jnp.concatenate([src, jnp.full((npad,), _N, jnp.int32)])
    dst_p3 = jnp.concatenate([dst, jnp.full((npad,), _N, jnp.int32)])

    ones_f = jnp.ones((_CHUNK, _F), jnp.float32)
    zeros_f = jnp.zeros((_RPT, _F), jnp.float32)
    idx2 = jnp.stack([src_d.reshape(_NS, _CPT, _CHUNK),
                      dst_p3.reshape(_NS, _CPT, _CHUNK)])
    dst_p = dst_p3.reshape(shape3)

    deg = _sc_degree(idx2, ones_f, zeros_f)
    h1, no16, ni16 = _tc_layer1(deg[0], deg[1], in_feat, W1)
    agg_p = _sc_msg_h(h1, src_g, dst_p, zeros_f)
    h2 = _tc_layer2(agg_p[0, :_N], agg_p[1, :_N], ni16,
                    b1.reshape(1, _F), W2, no16)
    out_p = _sc_msg_h(h2, src_g, dst_p, zeros_f)
    return _tc_final(out_p[0, :_N], out_p[1, :_N], ni16, b2.reshape(1, _C))


# speedup vs baseline: 2.3880x; 2.3880x over previous
import functools

import jax
import jax.numpy as jnp
from jax import lax
from jax.experimental import pallas as pl
from jax.experimental.pallas import tpu as pltpu
from jax.experimental.pallas import tpu_sc as plsc

_N = 10000
_E = 320000
_F = 128
_C = 64

_NC = 2
_NS = 16
_NW = _NC * _NS
_CHUNK = 128
_CPW = 80
_IBLK = 8
_EPAD = _NW * _CPW * _CHUNK
_ACC = 10112
_RPT = _ACC // _NS

_mesh = plsc.VectorSubcoreMesh(core_axis_name="c", subcore_axis_name="s")


_NCHUNK = _EPAD // _CHUNK
_CPT = _NCHUNK // _NS


def _sc_degree_body(idx_hbm, ones_hbm, zeros_hbm, deg_hbm,
                    idx_v, ones_v, acc):
    cid = lax.axis_index("c")
    sid = lax.axis_index("s")
    rows = pl.ds(sid * _RPT, _RPT)
    pltpu.sync_copy(zeros_hbm, acc.at[rows])
    pltpu.sync_copy(ones_hbm, ones_v)
    pltpu.sync_copy(idx_hbm.at[cid, sid], idx_v)
    plsc.subcore_barrier()

    def body(j, carry):
        pltpu.sync_copy(ones_v, acc.at[idx_v.at[j]], add=True)
        return carry

    lax.fori_loop(0, _CPT, body, 0)
    plsc.subcore_barrier()
    pltpu.sync_copy(acc.at[rows], deg_hbm.at[cid, rows])


_sc_degree = pl.kernel(
    _sc_degree_body,
    out_type=jax.ShapeDtypeStruct((_NC, _ACC, _F), jnp.float32),
    mesh=_mesh,
    scratch_types=[
        pltpu.VMEM((_CPT, _CHUNK), jnp.int32),
        pltpu.VMEM((_CHUNK, _F), jnp.float32),
        pltpu.VMEM_SHARED((_ACC, _F), jnp.float32),
    ],
)


def _sc_msg_body(feat, h_hbm, src_hbm, dst_hbm, zeros_hbm, out_hbm,
                 src_v, dst_v, msg_a, sem, acc):
    cid = lax.axis_index("c")
    sid = lax.axis_index("s")
    w = cid * _NS + sid
    rows = pl.ds(sid * _RPT, _RPT)
    pltpu.sync_copy(zeros_hbm, acc.at[rows])
    pltpu.sync_copy(src_hbm.at[w], src_v)
    pltpu.sync_copy(dst_hbm.at[w], dst_v)
    plsc.subcore_barrier()

    def body(j, carry):
        pltpu.async_copy(h_hbm.at[src_v.at[j]], msg_a, sem).wait()
        pltpu.sync_copy(msg_a, acc.at[dst_v.at[j]], add=True)
        return carry

    lax.fori_loop(0, _CPW, body, 0)
    plsc.subcore_barrier()
    pltpu.sync_copy(acc.at[rows], out_hbm.at[cid, rows])


def _make_sc_msg(feat):
    return pl.kernel(
        functools.partial(_sc_msg_body, feat),
        out_type=jax.ShapeDtypeStruct((_NC, _ACC, feat), jnp.float32),
        mesh=_mesh,
        scratch_types=[
            pltpu.VMEM((_CPW, _CHUNK), jnp.int32),
            pltpu.VMEM((_CPW, _CHUNK), jnp.int32),
            pltpu.VMEM((_CHUNK, feat), jnp.float32),
            pltpu.SemaphoreType.DMA,
            pltpu.VMEM_SHARED((_ACC, feat), jnp.float32),
        ],
    )


_sc_msg_h = _make_sc_msg(_F)


_BLK = 1000


def _tc1_body(dout, din, x, w1, h1, no16, ni16):
    no = lax.rsqrt(jnp.maximum(dout[..., :16], 1.0))
    ni = lax.rsqrt(jnp.maximum(din[..., :16], 1.0))
    no16[...] = no
    ni16[...] = ni
    h = jnp.dot(x[...], w1[...], preferred_element_type=jnp.float32,
                precision=lax.Precision.HIGHEST)
    h1[...] = h * no[:, 0:1]


def _tc_layer1(dout, din, x, w1):
    spec16 = pl.BlockSpec((_BLK, 16), lambda i: (i, 0))
    specF = pl.BlockSpec((_BLK, _F), lambda i: (i, 0))
    return pl.pallas_call(
        _tc1_body,
        grid=(_N // _BLK,),
        in_specs=[
            specF, specF, specF,
            pl.BlockSpec((_F, _F), lambda i: (0, 0)),
        ],
        out_specs=[specF, spec16, spec16],
        out_shape=[
            jax.ShapeDtypeStruct((_N, _F), jnp.float32),
            jax.ShapeDtypeStruct((_N, 16), jnp.float32),
            jax.ShapeDtypeStruct((_N, 16), jnp.float32),
        ],
    )(dout, din, x, w1)


def _tc2_body(a0, a1, ni16, b1, w2, no16, h2):
    agg = (a0[...] + a1[...]) * ni16[..., 0:1] + b1[...]
    h = jnp.maximum(agg, 0.0)
    h = jnp.dot(h, w2[...], preferred_element_type=jnp.float32,
                precision=lax.Precision.HIGHEST)
    h = h * no16[..., 0:1]
    h2[...] = jnp.concatenate([h, jnp.zeros_like(h)], axis=1)


def _tc_layer2(a0, a1, ni16, b1, w2, no16):
    spec16 = pl.BlockSpec((_BLK, 16), lambda i: (i, 0))
    return pl.pallas_call(
        _tc2_body,
        grid=(_N // _BLK,),
        in_specs=[
            pl.BlockSpec((_BLK, _F), lambda i: (i, 0)),
            pl.BlockSpec((_BLK, _F), lambda i: (i, 0)),
            spec16,
            pl.BlockSpec((1, _F), lambda i: (0, 0)),
            pl.BlockSpec((_F, _C), lambda i: (0, 0)),
            spec16,
        ],
        out_specs=pl.BlockSpec((_BLK, 2 * _C), lambda i: (i, 0)),
        out_shape=jax.ShapeDtypeStruct((_N, 2 * _C), jnp.float32),
    )(a0, a1, ni16, b1, w2, no16)


def _tc3_body(p0, p1, ni16, b2, out):
    agg = p0[..., :_C] + p1[..., :_C]
    out[...] = agg * ni16[..., 0:1] + b2[...]


def _tc_final(p0, p1, ni16, b2):
    spec16 = pl.BlockSpec((_BLK, 16), lambda i: (i, 0))
    return pl.pallas_call(
        _tc3_body,
        grid=(_N // _BLK,),
        in_specs=[
            pl.BlockSpec((_BLK, _F), lambda i: (i, 0)),
            pl.BlockSpec((_BLK, _F), lambda i: (i, 0)),
            spec16,
            pl.BlockSpec((1, _C), lambda i: (0, 0)),
        ],
        out_specs=pl.BlockSpec((_BLK, _C), lambda i: (i, 0)),
        out_shape=jax.ShapeDtypeStruct((_N, _C), jnp.float32),
    )(p0, p1, ni16, b2)


def kernel(in_feat, edge_index, W1, b1, W2, b2):
    src = edge_index[0].astype(jnp.int32)
    dst = edge_index[1].astype(jnp.int32)
    npad = _EPAD - _E
    shape3 = (_NW, _CPW, _CHUNK)
    iota_pad = lax.iota(jnp.int32, npad)
    pad_real = iota_pad % _N
    pad_dummy = _N + iota_pad % (_ACC - _N)
    src_g = jnp.concatenate([src, pad_real]).reshape(shape3)
    src_d = jnp.concatenate([src, pad_dummy])
    dst_p3 = jnp.concatenate([dst, pad_dummy])

    ones_f = jnp.ones((_CHUNK, _F), jnp.float32)
    zeros_f = jnp.zeros((_RPT, _F), jnp.float32)
    idx2 = jnp.stack([src_d.reshape(_NS, _CPT, _CHUNK),
                      dst_p3.reshape(_NS, _CPT, _CHUNK)])
    dst_p = dst_p3.reshape(shape3)

    deg = _sc_degree(idx2, ones_f, zeros_f)
    h1, no16, ni16 = _tc_layer1(deg[0], deg[1], in_feat, W1)
    agg_p = _sc_msg_h(h1, src_g, dst_p, zeros_f)
    h2 = _tc_layer2(agg_p[0, :_N], agg_p[1, :_N], ni16,
                    b1.reshape(1, _F), W2, no16)
    out_p = _sc_msg_h(h2, src_g, dst_p, zeros_f)
    return _tc_final(out_p[0, :_N], out_p[1, :_N], ni16, b2.reshape(1, _C))


# speedup vs baseline: 3.0612x; 1.2819x over previous
import functools

import jax
import jax.numpy as jnp
from jax import lax
from jax.experimental import pallas as pl
from jax.experimental.pallas import tpu as pltpu
from jax.experimental.pallas import tpu_sc as plsc

_N = 10000
_E = 320000
_F = 128
_C = 64

_NC = 2
_NS = 16
_NW = _NC * _NS
_CHUNK = 128
_CPW = 80
_IBLK = 8
_EPAD = _NW * _CPW * _CHUNK
_ACC = 10112
_RPT = _ACC // _NS

_mesh = plsc.VectorSubcoreMesh(core_axis_name="c", subcore_axis_name="s")


_NCHUNK = _EPAD // _CHUNK
_CPT = _NCHUNK // _NS


def _sc_degree_body(idx_hbm, ones_hbm, zeros_hbm, deg_hbm,
                    idx_v, ones_v, acc):
    cid = lax.axis_index("c")
    sid = lax.axis_index("s")
    rows = pl.ds(sid * _RPT, _RPT)
    pltpu.sync_copy(zeros_hbm, acc.at[rows])
    pltpu.sync_copy(ones_hbm, ones_v)
    pltpu.sync_copy(idx_hbm.at[cid, sid], idx_v)
    plsc.subcore_barrier()

    def body(j, carry):
        pltpu.sync_copy(ones_v, acc.at[idx_v.at[j]], add=True)
        return carry

    lax.fori_loop(0, _CPT, body, 0)
    plsc.subcore_barrier()
    pltpu.sync_copy(acc.at[rows], deg_hbm.at[cid, rows])


_sc_degree = pl.kernel(
    _sc_degree_body,
    out_type=jax.ShapeDtypeStruct((_NC, _ACC, _F), jnp.float32),
    mesh=_mesh,
    scratch_types=[
        pltpu.VMEM((_CPT, _CHUNK), jnp.int32),
        pltpu.VMEM((_CHUNK, _F), jnp.float32),
        pltpu.VMEM_SHARED((_ACC, _F), jnp.float32),
    ],
)


def _sc_msg_body(feat, h_hbm, src_hbm, dst_hbm, zeros_hbm, out_hbm,
                 src_v, dst_v, msg_v, sem, acc):
    cid = lax.axis_index("c")
    sid = lax.axis_index("s")
    w = cid * _NS + sid
    rows = pl.ds(sid * _RPT, _RPT)
    pltpu.sync_copy(zeros_hbm, acc.at[rows])
    pltpu.sync_copy(src_hbm.at[w], src_v)
    pltpu.sync_copy(dst_hbm.at[w, 0], dst_v.at[0])
    plsc.subcore_barrier()
    pltpu.async_copy(h_hbm.at[src_v.at[0]], msg_v.at[0], sem)

    def step(j, prefetch):
        if prefetch:
            nxt = j + 1
            nb = nxt // _IBLK

            @pl.when(nxt % _IBLK == 0)
            def _():
                pltpu.sync_copy(dst_hbm.at[w, nb], dst_v.at[nb % 2])

            pltpu.async_copy(h_hbm.at[src_v.at[nxt]], msg_v.at[nxt % 2], sem)
        pltpu.make_async_copy(h_hbm.at[src_v.at[0]], msg_v.at[j % 2],
                              sem).wait()
        pltpu.sync_copy(msg_v.at[j % 2],
                        acc.at[dst_v.at[(j // _IBLK) % 2, j % _IBLK]],
                        add=True)

    def body(j, carry):
        step(j, True)
        return carry

    lax.fori_loop(0, _CPW - 1, body, 0)
    step(_CPW - 1, False)
    plsc.subcore_barrier()
    pltpu.sync_copy(acc.at[rows], out_hbm.at[cid, rows])


def _make_sc_msg(feat):
    return pl.kernel(
        functools.partial(_sc_msg_body, feat),
        out_type=jax.ShapeDtypeStruct((_NC, _ACC, feat), jnp.float32),
        mesh=_mesh,
        scratch_types=[
            pltpu.VMEM((_CPW, _CHUNK), jnp.int32),
            pltpu.VMEM((2, _IBLK, _CHUNK), jnp.int32),
            pltpu.VMEM((2, _CHUNK, feat), jnp.float32),
            pltpu.SemaphoreType.DMA,
            pltpu.VMEM_SHARED((_ACC, feat), jnp.float32),
        ],
    )


_sc_msg_h = _make_sc_msg(_F)


_BLK = 1000


def _tc1_body(dout, din, x, w1, h1, no16, ni16):
    no = lax.rsqrt(jnp.maximum(dout[..., :16], 1.0))
    ni = lax.rsqrt(jnp.maximum(din[..., :16], 1.0))
    no16[...] = no
    ni16[...] = ni
    h = jnp.dot(x[...], w1[...], preferred_element_type=jnp.float32,
                precision=lax.Precision.HIGHEST)
    h1[...] = h * no[:, 0:1]


def _tc_layer1(dout, din, x, w1):
    spec16 = pl.BlockSpec((_BLK, 16), lambda i: (i, 0))
    specF = pl.BlockSpec((_BLK, _F), lambda i: (i, 0))
    return pl.pallas_call(
        _tc1_body,
        grid=(_N // _BLK,),
        in_specs=[
            specF, specF, specF,
            pl.BlockSpec((_F, _F), lambda i: (0, 0)),
        ],
        out_specs=[specF, spec16, spec16],
        out_shape=[
            jax.ShapeDtypeStruct((_N, _F), jnp.float32),
            jax.ShapeDtypeStruct((_N, 16), jnp.float32),
            jax.ShapeDtypeStruct((_N, 16), jnp.float32),
        ],
    )(dout, din, x, w1)


def _tc2_body(a0, a1, ni16, b1, w2, no16, h2):
    agg = (a0[...] + a1[...]) * ni16[..., 0:1] + b1[...]
    h = jnp.maximum(agg, 0.0)
    h = jnp.dot(h, w2[...], preferred_element_type=jnp.float32,
                precision=lax.Precision.HIGHEST)
    h = h * no16[..., 0:1]
    h2[...] = jnp.concatenate([h, jnp.zeros_like(h)], axis=1)


def _tc_layer2(a0, a1, ni16, b1, w2, no16):
    spec16 = pl.BlockSpec((_BLK, 16), lambda i: (i, 0))
    return pl.pallas_call(
        _tc2_body,
        grid=(_N // _BLK,),
        in_specs=[
            pl.BlockSpec((_BLK, _F), lambda i: (i, 0)),
            pl.BlockSpec((_BLK, _F), lambda i: (i, 0)),
            spec16,
            pl.BlockSpec((1, _F), lambda i: (0, 0)),
            pl.BlockSpec((_F, _C), lambda i: (0, 0)),
            spec16,
        ],
        out_specs=pl.BlockSpec((_BLK, 2 * _C), lambda i: (i, 0)),
        out_shape=jax.ShapeDtypeStruct((_N, 2 * _C), jnp.float32),
    )(a0, a1, ni16, b1, w2, no16)


def _tc3_body(p0, p1, ni16, b2, out):
    agg = p0[..., :_C] + p1[..., :_C]
    out[...] = agg * ni16[..., 0:1] + b2[...]


def _tc_final(p0, p1, ni16, b2):
    spec16 = pl.BlockSpec((_BLK, 16), lambda i: (i, 0))
    return pl.pallas_call(
        _tc3_body,
        grid=(_N // _BLK,),
        in_specs=[
            pl.BlockSpec((_BLK, _F), lambda i: (i, 0)),
            pl.BlockSpec((_BLK, _F), lambda i: (i, 0)),
            spec16,
            pl.BlockSpec((1, _C), lambda i: (0, 0)),
        ],
        out_specs=pl.BlockSpec((_BLK, _C), lambda i: (i, 0)),
        out_shape=jax.ShapeDtypeStruct((_N, _C), jnp.float32),
    )(p0, p1, ni16, b2)


def kernel(in_feat, edge_index, W1, b1, W2, b2):
    src = edge_index[0].astype(jnp.int32)
    dst = edge_index[1].astype(jnp.int32)
    npad = _EPAD - _E
    shape3 = (_NW, _CPW, _CHUNK)
    iota_pad = lax.iota(jnp.int32, npad)
    pad_real = iota_pad % _N
    pad_dummy = _N + iota_pad % (_ACC - _N)
    src_g = jnp.concatenate([src, pad_real]).reshape(shape3)
    src_d = jnp.concatenate([src, pad_dummy])
    dst_p3 = jnp.concatenate([dst, pad_dummy])

    ones_f = jnp.ones((_CHUNK, _F), jnp.float32)
    zeros_f = jnp.zeros((_RPT, _F), jnp.float32)
    idx2 = jnp.stack([src_d.reshape(_NS, _CPT, _CHUNK),
                      dst_p3.reshape(_NS, _CPT, _CHUNK)])
    dst_p = dst_p3.reshape(_NW, _CPW // _IBLK, _IBLK, _CHUNK)

    deg = _sc_degree(idx2, ones_f, zeros_f)
    h1, no16, ni16 = _tc_layer1(deg[0], deg[1], in_feat, W1)
    agg_p = _sc_msg_h(h1, src_g, dst_p, zeros_f)
    h2 = _tc_layer2(agg_p[0, :_N], agg_p[1, :_N], ni16,
                    b1.reshape(1, _F), W2, no16)
    out_p = _sc_msg_h(h2, src_g, dst_p, zeros_f)
    return _tc_final(out_p[0, :_N], out_p[1, :_N], ni16, b2.reshape(1, _C))


# speedup vs baseline: 3.1973x; 1.0445x over previous
import functools

import jax
import jax.numpy as jnp
from jax import lax
from jax.experimental import pallas as pl
from jax.experimental.pallas import tpu as pltpu
from jax.experimental.pallas import tpu_sc as plsc

_N = 10000
_E = 320000
_F = 128
_C = 64

_NC = 2
_NS = 16
_NW = _NC * _NS
_CHUNK = 128
_CPW = 80
_IBLK = 8
_EPAD = _NW * _CPW * _CHUNK
_ACC = 10112
_RPT = _ACC // _NS

_mesh = plsc.VectorSubcoreMesh(core_axis_name="c", subcore_axis_name="s")


_NCHUNK = _EPAD // _CHUNK
_CPT = _NCHUNK // _NS


def _sc_degree_body(idx_hbm, ones_hbm, zeros_hbm, deg_hbm,
                    idx_v, ones_v, acc):
    cid = lax.axis_index("c")
    sid = lax.axis_index("s")
    rows = pl.ds(sid * _RPT, _RPT)
    pltpu.sync_copy(zeros_hbm, acc.at[rows])
    pltpu.sync_copy(ones_hbm, ones_v)
    pltpu.sync_copy(idx_hbm.at[cid, sid], idx_v)
    plsc.subcore_barrier()

    def body(j, carry):
        pltpu.sync_copy(ones_v, acc.at[idx_v.at[j]], add=True)
        return carry

    lax.fori_loop(0, _CPT, body, 0)
    plsc.subcore_barrier()
    pltpu.sync_copy(acc.at[rows], deg_hbm.at[cid, rows])


_sc_degree = pl.kernel(
    _sc_degree_body,
    out_type=jax.ShapeDtypeStruct((_NC, _ACC, _F), jnp.float32),
    mesh=_mesh,
    scratch_types=[
        pltpu.VMEM((_CPT, _CHUNK), jnp.int32),
        pltpu.VMEM((_CHUNK, _F), jnp.float32),
        pltpu.VMEM_SHARED((_ACC, _F), jnp.float32),
    ],
)


def _sc_msg_body(feat, h_hbm, src_hbm, dst_hbm, zeros_hbm, out_hbm,
                 src_v, dst_v, msg_v, sem, acc):
    cid = lax.axis_index("c")
    sid = lax.axis_index("s")
    w = cid * _NS + sid
    rows = pl.ds(sid * _RPT, _RPT)
    pltpu.sync_copy(zeros_hbm, acc.at[rows])
    pltpu.sync_copy(src_hbm.at[w], src_v)
    pltpu.sync_copy(dst_hbm.at[w, 0], dst_v.at[0])
    plsc.subcore_barrier()
    pltpu.async_copy(h_hbm.at[src_v.at[0]], msg_v.at[0], sem)

    def step(j, prefetch):
        if prefetch:
            nxt = j + 1
            nb = nxt // _IBLK

            @pl.when(nxt % _IBLK == 0)
            def _():
                pltpu.sync_copy(dst_hbm.at[w, nb], dst_v.at[nb % 2])

            pltpu.async_copy(h_hbm.at[src_v.at[nxt]], msg_v.at[nxt % 2], sem)
        pltpu.make_async_copy(h_hbm.at[src_v.at[0]], msg_v.at[j % 2],
                              sem).wait()
        pltpu.sync_copy(msg_v.at[j % 2],
                        acc.at[dst_v.at[(j // _IBLK) % 2, j % _IBLK]],
                        add=True)

    def body(j, carry):
        step(j, True)
        return carry

    lax.fori_loop(0, _CPW - 1, body, 0)
    step(_CPW - 1, False)
    plsc.subcore_barrier()
    pltpu.sync_copy(acc.at[rows], out_hbm.at[cid, rows])


def _make_sc_msg(feat):
    return pl.kernel(
        functools.partial(_sc_msg_body, feat),
        out_type=jax.ShapeDtypeStruct((_NC, _ACC, feat), jnp.float32),
        mesh=_mesh,
        scratch_types=[
            pltpu.VMEM((_CPW, _CHUNK), jnp.int32),
            pltpu.VMEM((2, _IBLK, _CHUNK), jnp.int32),
            pltpu.VMEM((2, _CHUNK, feat), jnp.float32),
            pltpu.SemaphoreType.DMA,
            pltpu.VMEM_SHARED((_ACC, feat), jnp.float32),
        ],
    )


_sc_msg_h = _make_sc_msg(_F)


_BLK = 1000


def _tc1_body(dout, din, x, w1, h1, no16, ni16):
    no = lax.rsqrt(jnp.maximum(dout[0, :, :16].astype(jnp.float32), 1.0))
    ni = lax.rsqrt(jnp.maximum(din[0, :, :16].astype(jnp.float32), 1.0))
    no16[...] = no
    ni16[...] = ni
    h = jnp.dot(x[...], w1[...], preferred_element_type=jnp.float32,
                precision=lax.Precision.HIGHEST)
    h1[...] = h * no[:, 0:1]


def _tc_layer1(deg, x, w1):
    spec16 = pl.BlockSpec((_BLK, 16), lambda i: (i, 0))
    specF = pl.BlockSpec((_BLK, _F), lambda i: (i, 0))
    return pl.pallas_call(
        _tc1_body,
        grid=(_N // _BLK,),
        in_specs=[
            pl.BlockSpec((1, _BLK, _F), lambda i: (0, i, 0)),
            pl.BlockSpec((1, _BLK, _F), lambda i: (1, i, 0)),
            specF,
            pl.BlockSpec((_F, _F), lambda i: (0, 0)),
        ],
        out_specs=[specF, spec16, spec16],
        out_shape=[
            jax.ShapeDtypeStruct((_N, _F), jnp.float32),
            jax.ShapeDtypeStruct((_N, 16), jnp.float32),
            jax.ShapeDtypeStruct((_N, 16), jnp.float32),
        ],
    )(deg, deg, x, w1)


def _tc2_body(a0, a1, ni16, b1, w2, no16, h2):
    agg = (a0[0] + a1[0]) * ni16[..., 0:1] + b1[...]
    h = jnp.maximum(agg, 0.0)
    h = jnp.dot(h, w2[...], preferred_element_type=jnp.float32,
                precision=lax.Precision.HIGHEST)
    h = h * no16[..., 0:1]
    h2[...] = jnp.concatenate([h, jnp.zeros_like(h)], axis=1)


def _tc_layer2(a0, a1, ni16, b1, w2, no16):
    spec16 = pl.BlockSpec((_BLK, 16), lambda i: (i, 0))
    return pl.pallas_call(
        _tc2_body,
        grid=(_N // _BLK,),
        in_specs=[
            pl.BlockSpec((1, _BLK, _F), lambda i: (0, i, 0)),
            pl.BlockSpec((1, _BLK, _F), lambda i: (1, i, 0)),
            spec16,
            pl.BlockSpec((1, _F), lambda i: (0, 0)),
            pl.BlockSpec((_F, _C), lambda i: (0, 0)),
            spec16,
        ],
        out_specs=pl.BlockSpec((_BLK, 2 * _C), lambda i: (i, 0)),
        out_shape=jax.ShapeDtypeStruct((_N, 2 * _C), jnp.float32),
    )(a0, a1, ni16, b1, w2, no16)


def _tc3_body(p0, p1, ni16, b2, out):
    agg = p0[0, :, :_C] + p1[0, :, :_C]
    out[...] = agg * ni16[..., 0:1] + b2[...]


def _tc_final(p0, p1, ni16, b2):
    spec16 = pl.BlockSpec((_BLK, 16), lambda i: (i, 0))
    return pl.pallas_call(
        _tc3_body,
        grid=(_N // _BLK,),
        in_specs=[
            pl.BlockSpec((1, _BLK, _F), lambda i: (0, i, 0)),
            pl.BlockSpec((1, _BLK, _F), lambda i: (1, i, 0)),
            spec16,
            pl.BlockSpec((1, _C), lambda i: (0, 0)),
        ],
        out_specs=pl.BlockSpec((_BLK, _C), lambda i: (i, 0)),
        out_shape=jax.ShapeDtypeStruct((_N, _C), jnp.float32),
    )(p0, p1, ni16, b2)


def kernel(in_feat, edge_index, W1, b1, W2, b2):
    src = edge_index[0].astype(jnp.int32)
    dst = edge_index[1].astype(jnp.int32)
    npad = _EPAD - _E
    shape3 = (_NW, _CPW, _CHUNK)
    iota_pad = lax.iota(jnp.int32, npad)
    pad_real = iota_pad % _N
    pad_dummy = _N + iota_pad % (_ACC - _N)
    src_g = jnp.concatenate([src, pad_real]).reshape(shape3)
    src_d = jnp.concatenate([src, pad_dummy])
    dst_p3 = jnp.concatenate([dst, pad_dummy])

    ones_h = jnp.ones((_CHUNK, _F), jnp.float32)
    zeros_f = jnp.zeros((_RPT, _F), jnp.float32)
    idx2 = jnp.stack([src_d.reshape(_NS, _CPT, _CHUNK),
                      dst_p3.reshape(_NS, _CPT, _CHUNK)])
    dst_p = dst_p3.reshape(_NW, _CPW // _IBLK, _IBLK, _CHUNK)

    deg = _sc_degree(idx2, ones_h, zeros_f)
    h1, no16, ni16 = _tc_layer1(deg, in_feat, W1)
    agg_p = _sc_msg_h(h1, src_g, dst_p, zeros_f)
    h2 = _tc_layer2(agg_p, agg_p, ni16, b1.reshape(1, _F), W2, no16)
    out_p = _sc_msg_h(h2, src_g, dst_p, zeros_f)
    return _tc_final(out_p, out_p, ni16, b2.reshape(1, _C))


# speedup vs baseline: 3.2697x; 1.0226x over previous
import functools

import jax
import jax.numpy as jnp
from jax import lax
from jax.experimental import pallas as pl
from jax.experimental.pallas import tpu as pltpu
from jax.experimental.pallas import tpu_sc as plsc

_N = 10000
_E = 320000
_F = 128
_C = 64

_NC = 2
_NS = 16
_NW = _NC * _NS
_CHUNK = 128
_CPW = 80
_IBLK = 8
_EPAD = _NW * _CPW * _CHUNK
_ACC = 10112
_RPT = _ACC // _NS

_mesh = plsc.VectorSubcoreMesh(core_axis_name="c", subcore_axis_name="s")


_NCHUNK = _EPAD // _CHUNK
_CPT = _NCHUNK // _NS


def _sc_degree_body(idx_hbm, ones_hbm, zeros_hbm, deg_hbm,
                    idx_v, ones_v, acc):
    cid = lax.axis_index("c")
    sid = lax.axis_index("s")
    rows = pl.ds(sid * _RPT, _RPT)
    pltpu.sync_copy(zeros_hbm, acc.at[rows])
    pltpu.sync_copy(ones_hbm, ones_v)
    pltpu.sync_copy(idx_hbm.at[cid, sid], idx_v)
    plsc.subcore_barrier()

    def body(j, carry):
        pltpu.sync_copy(ones_v, acc.at[idx_v.at[j]], add=True)
        return carry

    lax.fori_loop(0, _CPT, body, 0)
    plsc.subcore_barrier()
    pltpu.sync_copy(acc.at[rows], deg_hbm.at[cid, rows])


_sc_degree = pl.kernel(
    _sc_degree_body,
    out_type=jax.ShapeDtypeStruct((_NC, _ACC, _F), jnp.float32),
    mesh=_mesh,
    scratch_types=[
        pltpu.VMEM((_CPT, _CHUNK), jnp.int32),
        pltpu.VMEM((_CHUNK, _F), jnp.float32),
        pltpu.VMEM_SHARED((_ACC, _F), jnp.float32),
    ],
)


def _sc_msg_body(feat, h_hbm, src_hbm, dst_hbm, zeros_hbm, out_hbm,
                 src_v, dst_v, msg_v, sem, acc):
    cid = lax.axis_index("c")
    sid = lax.axis_index("s")
    w = cid * _NS + sid
    rows = pl.ds(sid * _RPT, _RPT)
    pltpu.sync_copy(zeros_hbm, acc.at[rows])
    pltpu.sync_copy(src_hbm.at[w], src_v)
    pltpu.sync_copy(dst_hbm.at[w, 0], dst_v.at[0])
    plsc.subcore_barrier()
    pltpu.async_copy(h_hbm.at[src_v.at[0]], msg_v.at[0], sem)

    def step(j, prefetch):
        if prefetch:
            nxt = j + 1
            nb = nxt // _IBLK

            @pl.when(nxt % _IBLK == 0)
            def _():
                pltpu.sync_copy(dst_hbm.at[w, nb], dst_v.at[nb % 2])

            pltpu.async_copy(h_hbm.at[src_v.at[nxt]], msg_v.at[nxt % 2], sem)
        pltpu.make_async_copy(h_hbm.at[src_v.at[0]], msg_v.at[j % 2],
                              sem).wait()
        pltpu.sync_copy(msg_v.at[j % 2],
                        acc.at[dst_v.at[(j // _IBLK) % 2, j % _IBLK]],
                        add=True)

    def body(j, carry):
        step(j, True)
        return carry

    lax.fori_loop(0, _CPW - 1, body, 0)
    step(_CPW - 1, False)
    plsc.subcore_barrier()
    pltpu.sync_copy(acc.at[rows], out_hbm.at[cid, rows])


def _make_sc_msg(feat):
    return pl.kernel(
        functools.partial(_sc_msg_body, feat),
        out_type=jax.ShapeDtypeStruct((_NC, _ACC, feat), jnp.float32),
        mesh=_mesh,
        scratch_types=[
            pltpu.VMEM((_CPW, _CHUNK), jnp.int32),
            pltpu.VMEM((2, _IBLK, _CHUNK), jnp.int32),
            pltpu.VMEM((2, _CHUNK, feat), jnp.float32),
            pltpu.SemaphoreType.DMA,
            pltpu.VMEM_SHARED((_ACC, feat), jnp.float32),
        ],
    )


_sc_msg_h = _make_sc_msg(_F)


_BLK = 2000


def _tc1_body(dout, din, x, w1, h1, no16, ni16):
    no = lax.rsqrt(jnp.maximum(dout[0, :, :16].astype(jnp.float32), 1.0))
    ni = lax.rsqrt(jnp.maximum(din[0, :, :16].astype(jnp.float32), 1.0))
    no16[...] = no
    ni16[...] = ni
    h = jnp.dot(x[...], w1[...], preferred_element_type=jnp.float32,
                precision=lax.Precision.HIGHEST)
    h1[...] = h * no[:, 0:1]


def _tc_layer1(deg, x, w1):
    spec16 = pl.BlockSpec((_BLK, 16), lambda i: (i, 0))
    specF = pl.BlockSpec((_BLK, _F), lambda i: (i, 0))
    return pl.pallas_call(
        _tc1_body,
        grid=(_N // _BLK,),
        in_specs=[
            pl.BlockSpec((1, _BLK, _F), lambda i: (0, i, 0)),
            pl.BlockSpec((1, _BLK, _F), lambda i: (1, i, 0)),
            specF,
            pl.BlockSpec((_F, _F), lambda i: (0, 0)),
        ],
        out_specs=[specF, spec16, spec16],
        out_shape=[
            jax.ShapeDtypeStruct((_N, _F), jnp.float32),
            jax.ShapeDtypeStruct((_N, 16), jnp.float32),
            jax.ShapeDtypeStruct((_N, 16), jnp.float32),
        ],
    )(deg, deg, x, w1)


def _tc2_body(a0, a1, ni16, b1, w2, no16, h2):
    agg = (a0[0] + a1[0]) * ni16[..., 0:1] + b1[...]
    h = jnp.maximum(agg, 0.0)
    h = jnp.dot(h, w2[...], preferred_element_type=jnp.float32,
                precision=lax.Precision.HIGHEST)
    h = h * no16[..., 0:1]
    h2[...] = jnp.concatenate([h, jnp.zeros_like(h)], axis=1)


def _tc_layer2(a0, a1, ni16, b1, w2, no16):
    spec16 = pl.BlockSpec((_BLK, 16), lambda i: (i, 0))
    return pl.pallas_call(
        _tc2_body,
        grid=(_N // _BLK,),
        in_specs=[
            pl.BlockSpec((1, _BLK, _F), lambda i: (0, i, 0)),
            pl.BlockSpec((1, _BLK, _F), lambda i: (1, i, 0)),
            spec16,
            pl.BlockSpec((1, _F), lambda i: (0, 0)),
            pl.BlockSpec((_F, _C), lambda i: (0, 0)),
            spec16,
        ],
        out_specs=pl.BlockSpec((_BLK, 2 * _C), lambda i: (i, 0)),
        out_shape=jax.ShapeDtypeStruct((_N, 2 * _C), jnp.float32),
    )(a0, a1, ni16, b1, w2, no16)


def _tc3_body(p0, p1, ni16, b2, out):
    agg = p0[0, :, :_C] + p1[0, :, :_C]
    out[...] = agg * ni16[..., 0:1] + b2[...]


def _tc_final(p0, p1, ni16, b2):
    spec16 = pl.BlockSpec((_BLK, 16), lambda i: (i, 0))
    return pl.pallas_call(
        _tc3_body,
        grid=(_N // _BLK,),
        in_specs=[
            pl.BlockSpec((1, _BLK, _F), lambda i: (0, i, 0)),
            pl.BlockSpec((1, _BLK, _F), lambda i: (1, i, 0)),
            spec16,
            pl.BlockSpec((1, _C), lambda i: (0, 0)),
        ],
        out_specs=pl.BlockSpec((_BLK, _C), lambda i: (i, 0)),
        out_shape=jax.ShapeDtypeStruct((_N, _C), jnp.float32),
    )(p0, p1, ni16, b2)


def kernel(in_feat, edge_index, W1, b1, W2, b2):
    src = edge_index[0].astype(jnp.int32)
    dst = edge_index[1].astype(jnp.int32)
    npad = _EPAD - _E
    shape3 = (_NW, _CPW, _CHUNK)
    iota_pad = lax.iota(jnp.int32, npad)
    pad_real = iota_pad % _N
    pad_dummy = _N + iota_pad % (_ACC - _N)
    src_g = jnp.concatenate([src, pad_real]).reshape(shape3)
    src_d = jnp.concatenate([src, pad_dummy])
    dst_p3 = jnp.concatenate([dst, pad_dummy])

    ones_h = jnp.ones((_CHUNK, _F), jnp.float32)
    zeros_f = jnp.zeros((_RPT, _F), jnp.float32)
    idx2 = jnp.stack([src_d.reshape(_NS, _CPT, _CHUNK),
                      dst_p3.reshape(_NS, _CPT, _CHUNK)])
    dst_p = dst_p3.reshape(_NW, _CPW // _IBLK, _IBLK, _CHUNK)

    deg = _sc_degree(idx2, ones_h, zeros_f)
    h1, no16, ni16 = _tc_layer1(deg, in_feat, W1)
    agg_p = _sc_msg_h(h1, src_g, dst_p, zeros_f)
    h2 = _tc_layer2(agg_p, agg_p, ni16, b1.reshape(1, _F), W2, no16)
    out_p = _sc_msg_h(h2, src_g, dst_p, zeros_f)
    return _tc_final(out_p, out_p, ni16, b2.reshape(1, _C))
